# baseline (speedup 1.0000x reference)
import functools

import jax
import jax.numpy as jnp
from jax import lax
from jax.experimental import pallas as pl
from jax.experimental.pallas import tpu as pltpu
from jax.experimental.pallas import tpu_sc as plsc

NC = 2
NS = 16
NW = NC * NS
RPB = 4
NBUF = 8
K = 4


def _emb_body(table_hbm, idx_hbm, out_hbm, idx_v, rows_v, gsem, wsem):
    wid = lax.axis_index("s") * NC + lax.axis_index("c")
    rows_per_tile = idx_v.shape[0]
    n_blocks = rows_per_tile // RPB
    n_outer = n_blocks // NBUF
    row0 = wid * rows_per_tile
    pltpu.sync_copy(idx_hbm.at[pl.ds(row0, rows_per_tile)], idx_v)

    def fire_gathers(c, b):
        for i in range(RPB):
            pltpu.async_copy(table_hbm.at[idx_v.at[c * RPB + i]],
                             rows_v.at[b, i], gsem.at[b])

    def wait_gathers(c, b):
        for i in range(RPB):
            pltpu.make_async_copy(table_hbm.at[idx_v.at[c * RPB + i]],
                                  rows_v.at[b, i], gsem.at[b]).wait()

    def step(c, b, first_outer, last_outer):
        wait_gathers(c, b)
        pltpu.async_copy(rows_v.at[b], out_hbm.at[pl.ds(row0 + c * RPB, RPB)],
                         wsem.at[b])
        if not (last_outer and b >= NBUF - K):
            b2 = (b + K) % NBUF
            if not (first_outer and b < NBUF - K):
                pltpu.make_async_copy(
                    rows_v.at[b2], out_hbm.at[pl.ds(0, RPB)],
                    wsem.at[b2]).wait()
            fire_gathers(c + K, b2)

    for b in range(K):
        fire_gathers(b, b)

    for b in range(NBUF):
        step(b, b, True, False)

    def outer(g, carry):
        for b in range(NBUF):
            step(g * NBUF + b, b, False, False)
        return carry

    lax.fori_loop(1, n_outer - 1, outer, 0)

    for b in range(NBUF):
        step((n_outer - 1) * NBUF + b, b, False, True)

    for b in range(NBUF):
        pltpu.make_async_copy(
            rows_v.at[b], out_hbm.at[pl.ds(0, RPB)], wsem.at[b]).wait()


def kernel(input, weight):
    B, F = input.shape
    D = weight.shape[1]
    rows_per_tile = B // NW

    mesh = plsc.VectorSubcoreMesh(core_axis_name="c", subcore_axis_name="s")
    k = functools.partial(
        pl.kernel,
        mesh=mesh,
        compiler_params=pltpu.CompilerParams(use_tc_tiling_on_sc=False),
        out_type=jax.ShapeDtypeStruct((B, F, D), weight.dtype),
        scratch_types=[
            pltpu.VMEM((rows_per_tile, F), jnp.int32),
            pltpu.VMEM((NBUF, RPB, F, D), jnp.float32),
            pltpu.SemaphoreType.DMA((NBUF,)),
            pltpu.SemaphoreType.DMA((NBUF,)),
        ],
    )(_emb_body)
    return k(weight, input)
